# baseline (speedup 1.0000x reference)
import functools
import jax
import jax.numpy as jnp
import numpy as np
from jax import lax
from jax.experimental import pallas as pl
from jax.experimental.pallas import tpu as pltpu
from jax.experimental.pallas import tpu_sc as plsc

_B = 4096
_NS = 26
_ND = 13
_V = 100000
_D = 4
_NF = _NS + _ND
_EPS = 0.001
_PAD = 128

_fi, _fj = np.triu_indices(_NF, k=1)
_FI = jnp.asarray(_fi, dtype=jnp.int32)
_FJ = jnp.asarray(_fj, dtype=jnp.int32)

_NW = 32
_ROWS = _B * _NS
_CHUNK = 128
_RPW = _ROWS // _NW
_CPW = _RPW // _CHUNK


@functools.lru_cache(maxsize=None)
def _make_sc_gather():
    return functools.partial(
        pl.kernel,
        out_type=jax.ShapeDtypeStruct((_ROWS, _D), jnp.float32),
        mesh=plsc.VectorSubcoreMesh(core_axis_name="c", subcore_axis_name="s"),
        scratch_types=[
            pltpu.VMEM((_CPW, _CHUNK), jnp.int32),
            pltpu.VMEM((_RPW, _D), jnp.float32),
            pltpu.SemaphoreType.DMA,
        ],
        compiler_params=pltpu.CompilerParams(use_tc_tiling_on_sc=False),
    )(_sc_gather_body)


def _sc_gather_body(table_hbm, idx_hbm, out_hbm, idx_v, rows_v, sem):
    wid = lax.axis_index("s") * 2 + lax.axis_index("c")
    pltpu.sync_copy(idx_hbm.at[wid], idx_v)
    for half in range(2):
        cps = []
        for j in range(13):
            jj = half * 13 + j
            cps.append(
                pltpu.async_copy(
                    table_hbm.at[idx_v.at[jj]],
                    rows_v.at[pl.ds(jj * _CHUNK, _CHUNK)],
                    sem,
                )
            )
        for cp in cps:
            cp.wait()
    pltpu.sync_copy(rows_v, out_hbm.at[pl.ds(wid * _RPW, _RPW)])


def _dotT(a, b):
    return lax.dot_general(
        a, b, (((0,), (0,)), ((), ())),
        preferred_element_type=jnp.float32,
        precision=lax.Precision.HIGHEST,
    )


def _tc_body(es_ref, xd_ref, w_ref, bmat_ref, out_ref):
    dp = xd_ref[...] * w_ref[...]
    e = [es_ref[d] + dp for d in range(_D)]
    s = _dotT(e[0], e[0])
    for d in range(1, _D):
        s = s + _dotT(e[d], e[d])
    m2 = s * (1.0 / _B)
    q = None
    for d in range(_D):
        for d2 in range(d, _D):
            g = e[d] * e[d2]
            t = _dotT(g, g)
            t = t if d == d2 else t * 2.0
            q = t if q is None else q + t
    var = q * (1.0 / _B) - m2 * m2
    w = jnp.tanh(bmat_ref[...]) * lax.rsqrt(var + _EPS)
    c = jnp.sum(w * m2)
    acc = None
    for d in range(_D):
        a = jnp.dot(e[d], w, preferred_element_type=jnp.float32,
                    precision=lax.Precision.HIGHEST) * e[d]
        acc = a if acc is None else acc + a
    out_ref[...] = jnp.sum(acc, axis=1, keepdims=True) - c


_tc_call = pl.pallas_call(
    _tc_body,
    out_shape=jax.ShapeDtypeStruct((_B, 1), jnp.float32),
)


def kernel(X, emb_tables, weight, beta):
    sparse_idx = X[:, :_NS].astype(jnp.int32)
    idx3d = (sparse_idx + jnp.arange(_NS, dtype=jnp.int32)[None, :] * _V
             ).reshape(_NW, _CPW, _CHUNK)
    table = emb_tables.reshape(_NS * _V, _D)
    emb_flat = _make_sc_gather()(table, idx3d)
    es = emb_flat.reshape(_B, _NS, _D).transpose(2, 0, 1)
    es_pad = jnp.pad(es, ((0, 0), (0, 0), (0, _PAD - _NS)))
    xd_pad = jnp.zeros((_B, _PAD), jnp.float32).at[:, _NS:_NF].set(X[:, _NS:])
    wrow = jnp.zeros((1, _PAD), jnp.float32).at[0, _NS:_NF].set(weight[:, 0])
    bmat = jnp.zeros((_PAD, _PAD), jnp.float32).at[_FI, _FJ].set(beta)
    return _tc_call(es_pad, xd_pad, wrow, bmat)


# speedup vs baseline: 21.3381x; 21.3381x over previous
import functools
import jax
import jax.numpy as jnp
import numpy as np
from jax import lax
from jax.experimental import pallas as pl
from jax.experimental.pallas import tpu as pltpu
from jax.experimental.pallas import tpu_sc as plsc

_B = 4096
_NS = 26
_ND = 13
_V = 100000
_D = 4
_NF = _NS + _ND
_EPS = 0.001
_PAD = 128

_fi, _fj = np.triu_indices(_NF, k=1)
_FI = _fi.astype(np.int32)
_FJ = _fj.astype(np.int32)

_NW = 32
_NWORDS = _B * _NS * _D
_CHUNK = 128
_WPW = _NWORDS // _NW
_CPW = _WPW // _CHUNK
_FIRE = 13


@functools.lru_cache(maxsize=None)
def _make_sc_gather():
    return functools.partial(
        pl.kernel,
        out_type=jax.ShapeDtypeStruct((_NWORDS,), jnp.float32),
        mesh=plsc.VectorSubcoreMesh(core_axis_name="c", subcore_axis_name="s"),
        scratch_types=[
            pltpu.VMEM((_CPW, _CHUNK), jnp.int32),
            pltpu.VMEM((_WPW,), jnp.float32),
            pltpu.SemaphoreType.DMA,
        ],
        compiler_params=pltpu.CompilerParams(use_tc_tiling_on_sc=False),
    )(_sc_gather_body)


def _sc_gather_body(table_hbm, idx_hbm, out_hbm, idx_v, rows_v, sem):
    wid = lax.axis_index("s") * 2 + lax.axis_index("c")
    pltpu.sync_copy(idx_hbm.at[wid], idx_v)
    for g in range(_CPW // _FIRE):
        cps = []
        for j in range(_FIRE):
            jj = g * _FIRE + j
            cps.append(
                pltpu.async_copy(
                    table_hbm.at[idx_v.at[jj]],
                    rows_v.at[pl.ds(jj * _CHUNK, _CHUNK)],
                    sem,
                )
            )
        for cp in cps:
            cp.wait()
    pltpu.sync_copy(rows_v, out_hbm.at[pl.ds(wid * _WPW, _WPW)])


def _dotT(a, b):
    return lax.dot_general(
        a, b, (((0,), (0,)), ((), ())),
        preferred_element_type=jnp.float32,
        precision=lax.Precision.HIGHEST,
    )


def _tc_body(es_ref, xd_ref, w_ref, bmat_ref, out_ref):
    dp = xd_ref[...] * w_ref[...]
    e = [es_ref[d] + dp for d in range(_D)]
    s = _dotT(e[0], e[0])
    for d in range(1, _D):
        s = s + _dotT(e[d], e[d])
    m2 = s * (1.0 / _B)
    q = None
    for d in range(_D):
        for d2 in range(d, _D):
            g = e[d] * e[d2]
            t = _dotT(g, g)
            t = t if d == d2 else t * 2.0
            q = t if q is None else q + t
    var = q * (1.0 / _B) - m2 * m2
    w = jnp.tanh(bmat_ref[...]) * lax.rsqrt(var + _EPS)
    c = jnp.sum(w * m2)
    acc = None
    for d in range(_D):
        a = jnp.dot(e[d], w, preferred_element_type=jnp.float32,
                    precision=lax.Precision.HIGHEST) * e[d]
        acc = a if acc is None else acc + a
    out_ref[...] = jnp.sum(acc, axis=1, keepdims=True) - c


_tc_call = pl.pallas_call(
    _tc_body,
    out_shape=jax.ShapeDtypeStruct((_B, 1), jnp.float32),
    compiler_params=pltpu.CompilerParams(vmem_limit_bytes=100 * 1024 * 1024),
)


def kernel(X, emb_tables, weight, beta):
    sparse_idx = X[:, :_NS].astype(jnp.int32)
    table_flat = emb_tables.transpose(0, 2, 1).reshape(_NS * _D * _V)
    widx = (sparse_idx[None, :, :]
            + (jnp.arange(_D, dtype=jnp.int32) * _V)[:, None, None]
            + (jnp.arange(_NS, dtype=jnp.int32) * (_D * _V))[None, None, :]
            ).reshape(_NW, _CPW, _CHUNK)
    es_flat = _make_sc_gather()(table_flat, widx)
    es = es_flat.reshape(_D, _B, _NS)
    es_pad = jnp.pad(es, ((0, 0), (0, 0), (0, _PAD - _NS)))
    xd_pad = jnp.zeros((_B, _PAD), jnp.float32).at[:, _NS:_NF].set(X[:, _NS:])
    wrow = jnp.zeros((1, _PAD), jnp.float32).at[0, _NS:_NF].set(weight[:, 0])
    bmat = jnp.zeros((_PAD, _PAD), jnp.float32).at[_FI, _FJ].set(beta)
    return _tc_call(es_pad, xd_pad, wrow, bmat)
